# Initial kernel scaffold; baseline (speedup 1.0000x reference)
import jax
import jax.numpy as jnp
from jax.experimental import pallas as pl


def kernel(x, emb, W1, b1, W2, b2, W3, b3):
    raise NotImplementedError("write your pallas kernel here")


# speedup vs baseline: 1.9466x; 1.9466x over previous
import functools

import jax
import jax.numpy as jnp
from jax import lax
from jax.experimental import pallas as pl
from jax.experimental.pallas import tpu as pltpu
from jax.experimental.pallas import tpu_sc as plsc

N = 1024
W = 5
D = 64
K = 20
H = 128
B = 256

_NB = 8
_NW = 32
_ROWS_PER_W = (N * K) // _NW
_CHUNK = 64
_NCHUNK = _ROWS_PER_W // _CHUNK


def _topk_embproj_body(emb_ref, w1b_ref, b1_ref, topk_ref, proj_ref):
    emb = emb_ref[...]
    nrm = jnp.sqrt(jnp.sum(emb * emb, axis=1, keepdims=True))
    norm = emb / (nrm + 1e-12)
    sim = lax.dot_general(norm, norm, (((1,), (1,)), ((), ())),
                          preferred_element_type=jnp.float32)
    row = lax.broadcasted_iota(jnp.int32, (N, N), 0)
    col = lax.broadcasted_iota(jnp.int32, (N, N), 1)
    sim = sim - jnp.where(row == col, jnp.float32(1e9), jnp.float32(0.0))
    cols = []
    for _ in range(K):
        m = jnp.max(sim, axis=1, keepdims=True)
        cand = jnp.where(sim == m, col, jnp.int32(N))
        idxk = jnp.min(cand, axis=1, keepdims=True)
        cols.append(idxk)
        sim = jnp.where(col == idxk, jnp.float32(-jnp.inf), sim)
    topk_ref[...] = jnp.concatenate(cols, axis=1)
    proj_ref[...] = jnp.dot(emb, w1b_ref[...],
                            preferred_element_type=jnp.float32) + b1_ref[...][None, :]


def _topk_embproj(emb, W1b, b1):
    return pl.pallas_call(
        _topk_embproj_body,
        out_shape=(jax.ShapeDtypeStruct((N, K), jnp.int32),
                   jax.ShapeDtypeStruct((N, H), jnp.float32)),
    )(emb, W1b, b1)


def _sc_gather(xt, idx_flat):
    mesh = plsc.VectorSubcoreMesh(core_axis_name="c", subcore_axis_name="s")

    @functools.partial(
        pl.kernel,
        mesh=mesh,
        out_type=jax.ShapeDtypeStruct((N * K, W * B), jnp.float32),
        scratch_types=[
            pltpu.VMEM((_NCHUNK, _CHUNK), jnp.int32),
            pltpu.VMEM((_CHUNK, W * B), jnp.float32),
            pltpu.SemaphoreType.DMA,
        ],
    )
    def k(table_hbm, idx_hbm, out_hbm, idx_v, rows_v, sem):
        wid = lax.axis_index("s") * 2 + lax.axis_index("c")
        base = wid * _ROWS_PER_W
        for c in range(_NCHUNK):
            pltpu.sync_copy(idx_hbm.at[pl.ds(base + c * _CHUNK, _CHUNK)],
                            idx_v.at[c])
        for c in range(_NCHUNK):
            pltpu.async_copy(table_hbm.at[idx_v.at[c]], rows_v, sem).wait()
            pltpu.sync_copy(rows_v,
                            out_hbm.at[pl.ds(base + c * _CHUNK, _CHUNK)])

    return k(xt, idx_flat)


def _mlp_body(g_ref, proj_ref, w1p_ref, w2_ref, b2_ref, w3_ref, b3_ref, out_ref):
    w1p = w1p_ref[...]
    w2 = w2_ref[...]
    b2 = b2_ref[...]
    w3 = w3_ref[...]
    b3 = b3_ref[0]
    for n in range(_NB):
        g = g_ref[n]
        h1 = lax.dot_general(g, w1p, (((0,), (0,)), ((), ())),
                             preferred_element_type=jnp.float32)
        h1 = jnp.maximum(h1 + proj_ref[n][None, :], 0.0)
        h2 = jnp.maximum(jnp.dot(h1, w2, preferred_element_type=jnp.float32)
                         + b2[None, :], 0.0)
        o = lax.dot_general(w3, h2, (((0,), (1,)), ((), ())),
                            preferred_element_type=jnp.float32)
        out_ref[n, :] = o[0] + b3


def _mlp(gathered, proj, W1p, W2, b2, W3, b3):
    return pl.pallas_call(
        _mlp_body,
        grid=(N // _NB,),
        in_specs=[
            pl.BlockSpec((_NB, K * W, B), lambda i: (i, 0, 0)),
            pl.BlockSpec((_NB, H), lambda i: (i, 0)),
            pl.BlockSpec((K * W, H), lambda i: (0, 0)),
            pl.BlockSpec((H, H // 2), lambda i: (0, 0)),
            pl.BlockSpec((H // 2,), lambda i: (0,)),
            pl.BlockSpec((H // 2, 1), lambda i: (0, 0)),
            pl.BlockSpec(memory_space=pltpu.SMEM),
        ],
        out_specs=pl.BlockSpec((_NB, B), lambda i: (i, 0)),
        out_shape=jax.ShapeDtypeStruct((N, B), jnp.float32),
    )(gathered, proj, W1p, W2, b2, W3, b3)


def kernel(x, emb, W1, b1, W2, b2, W3, b3):
    W1b = W1[K * W:]
    W1p = W1[:K * W].reshape(W, K, H).transpose(1, 0, 2).reshape(K * W, H)
    xt = jnp.transpose(x, (2, 1, 0)).reshape(N, W * B)

    topk, proj = _topk_embproj(emb, W1b, b1)
    gathered = _sc_gather(xt, topk.reshape(-1))
    gathered = gathered.reshape(N, K * W, B)
    outT = _mlp(gathered, proj, W1p, W2, b2, W3, b3)
    return outT.T
